# baseline (device time: 51862 ns/iter reference)
import jax
import jax.numpy as jnp
from jax import lax
from jax.experimental import pallas as pl
from jax.experimental.pallas import tpu as pltpu

N_DEV = 8
N_SRC = 4
B = 2
SQ = 128
SKV_PER = 128
H_PER = 4
DH = 64
D_MODEL = 512
QH = 64


def kernel(x, Wq, K_ext, V_ext, Wo):
    def body(x_ref, wq_ref, k_ref, v_ref, wo_ref, out_ref,
             sendbuf, kvbuf, pbuf, acc, ctx_sc,
             kv_send_sems, kv_recv_sems, p_send_sems, p_recv_sems):
        my_id = lax.axis_index("i")

        barrier_sem = pltpu.get_barrier_semaphore()
        for peer in range(N_DEV):
            @pl.when(my_id != peer)
            def _(peer=peer):
                pl.semaphore_signal(
                    barrier_sem, inc=1,
                    device_id=(peer,), device_id_type=pl.DeviceIdType.MESH,
                )
        pl.semaphore_wait(barrier_sem, N_DEV - 1)

        for c in range(N_SRC):
            s = 2 * c

            @pl.when(my_id == s)
            def _(c=c, s=s):
                kvbuf[c, 0] = k_ref[:, :, 4 * s:4 * s + 4, :].astype(jnp.bfloat16)
                kvbuf[c, 1] = v_ref[:, :, 4 * s:4 * s + 4, :].astype(jnp.bfloat16)
                for dest in range(N_DEV):
                    if dest == s:
                        continue
                    sendbuf[dest, 0] = k_ref[:, :, 4 * dest:4 * dest + 4, :].astype(jnp.bfloat16)
                    sendbuf[dest, 1] = v_ref[:, :, 4 * dest:4 * dest + 4, :].astype(jnp.bfloat16)
                    pltpu.make_async_remote_copy(
                        src_ref=sendbuf.at[dest],
                        dst_ref=kvbuf.at[c],
                        send_sem=kv_send_sems.at[dest],
                        recv_sem=kv_recv_sems.at[c],
                        device_id=(dest,),
                        device_id_type=pl.DeviceIdType.MESH,
                    ).start()

        xv = x_ref[...].reshape(B * SQ, D_MODEL).astype(jnp.bfloat16)
        wq = wq_ref[...].astype(jnp.bfloat16)
        q = jnp.dot(xv, wq, preferred_element_type=jnp.float32)
        q = q.reshape(B, SQ, H_PER, DH).astype(jnp.bfloat16)

        for c in range(N_SRC):
            @pl.when(my_id != 2 * c)
            def _(c=c):
                pltpu.make_async_remote_copy(
                    src_ref=sendbuf.at[0],
                    dst_ref=kvbuf.at[c],
                    send_sem=kv_send_sems.at[0],
                    recv_sem=kv_recv_sems.at[c],
                    device_id=(0,),
                    device_id_type=pl.DeviceIdType.MESH,
                ).wait_recv()

        kall = kvbuf[:, 0]
        vall = kvbuf[:, 1]
        for qh in range(2):
            qs = q[:, qh * QH:(qh + 1) * QH]
            ks = jnp.concatenate(
                [kall[c, :, qh * QH:(qh + 1) * QH] for c in range(N_SRC)], axis=1
            )
            vs = jnp.concatenate(
                [vall[c, :, qh * QH:(qh + 1) * QH] for c in range(N_SRC)], axis=1
            )
            for h in range(H_PER):
                qhh = qs[:, :, h, :]
                khh = ks[:, :, h, :]
                vhh = vs[:, :, h, :]
                sc = lax.dot_general(
                    qhh, khh, (((2,), (2,)), ((0,), (0,))),
                    preferred_element_type=jnp.float32,
                ) * 0.125
                m = jnp.max(sc, axis=-1, keepdims=True)
                e = jnp.exp(sc - m)
                p = (e / jnp.sum(e, axis=-1, keepdims=True)).astype(jnp.bfloat16)
                cx = lax.dot_general(
                    p, vhh, (((2,), (1,)), ((0,), (0,))),
                    preferred_element_type=jnp.float32,
                )
                ctx_sc[:, qh * QH:(qh + 1) * QH, h, :] = cx

        ctxf = ctx_sc[...].astype(jnp.bfloat16).reshape(B, SQ, H_PER * DH)
        wo = wo_ref[...].astype(jnp.bfloat16)
        partial = jnp.stack(
            [jnp.dot(ctxf[b], wo, preferred_element_type=jnp.float32)
             for b in range(B)], axis=0)
        pbuf[...] = partial.astype(jnp.bfloat16)

        for s in range(N_DEV):
            @pl.when(my_id == s)
            def _(s=s):
                acc[s] = pbuf[...]
                for dest in range(N_DEV):
                    if dest == s:
                        continue
                    pltpu.make_async_remote_copy(
                        src_ref=pbuf,
                        dst_ref=acc.at[s],
                        send_sem=p_send_sems.at[dest],
                        recv_sem=p_recv_sems.at[s],
                        device_id=(dest,),
                        device_id_type=pl.DeviceIdType.MESH,
                    ).start()

        for src in range(N_DEV):
            @pl.when(my_id != src)
            def _(src=src):
                pltpu.make_async_remote_copy(
                    src_ref=pbuf,
                    dst_ref=acc.at[src],
                    send_sem=p_send_sems.at[0],
                    recv_sem=p_recv_sems.at[src],
                    device_id=(0,),
                    device_id_type=pl.DeviceIdType.MESH,
                ).wait_recv()

        total = acc[0].astype(jnp.float32)
        for s in range(1, N_DEV):
            total = total + acc[s].astype(jnp.float32)
        out_ref[...] = total

        for dest in range(N_DEV):
            @pl.when(jnp.logical_and(my_id % 2 == 0, my_id != dest))
            def _(dest=dest):
                pltpu.make_async_remote_copy(
                    src_ref=sendbuf.at[dest],
                    dst_ref=kvbuf.at[0],
                    send_sem=kv_send_sems.at[dest],
                    recv_sem=kv_recv_sems.at[0],
                    device_id=(0,),
                    device_id_type=pl.DeviceIdType.MESH,
                ).wait_send()

            @pl.when(my_id != dest)
            def _(dest=dest):
                pltpu.make_async_remote_copy(
                    src_ref=pbuf,
                    dst_ref=acc.at[0],
                    send_sem=p_send_sems.at[dest],
                    recv_sem=p_recv_sems.at[0],
                    device_id=(0,),
                    device_id_type=pl.DeviceIdType.MESH,
                ).wait_send()

    out_shape = jax.ShapeDtypeStruct((B, SQ, D_MODEL), jnp.float32)
    return pl.pallas_call(
        body,
        out_shape=out_shape,
        in_specs=[pl.BlockSpec(memory_space=pltpu.VMEM)] * 5,
        out_specs=pl.BlockSpec(memory_space=pltpu.VMEM),
        scratch_shapes=[
            pltpu.VMEM((N_DEV, 2, B, SKV_PER, H_PER, DH), jnp.bfloat16),
            pltpu.VMEM((N_SRC, 2, B, SKV_PER, H_PER, DH), jnp.bfloat16),
            pltpu.VMEM((B, SQ, D_MODEL), jnp.bfloat16),
            pltpu.VMEM((N_DEV, B, SQ, D_MODEL), jnp.bfloat16),
            pltpu.VMEM((B, SQ, H_PER, DH), jnp.float32),
            pltpu.SemaphoreType.DMA((N_DEV,)),
            pltpu.SemaphoreType.DMA((N_SRC,)),
            pltpu.SemaphoreType.DMA((N_DEV,)),
            pltpu.SemaphoreType.DMA((N_DEV,)),
        ],
        compiler_params=pltpu.CompilerParams(collective_id=0),
    )(x, Wq, K_ext, V_ext, Wo)
